# baseline (speedup 1.0000x reference)
import functools

import jax
import jax.numpy as jnp
from jax import lax
from jax.experimental import pallas as pl
from jax.experimental.pallas import tpu as pltpu
from jax.experimental.pallas import tpu_sc as plsc

V_SIZE = 100000
E_SIZE = 64
BATCH = 1024
HIST = 20

NUM_WORKERS = 32
B_PER_W = BATCH // NUM_WORKERS
IDX_PER_W = B_PER_W * HIST
IDX_CHUNK = 128
N_CHUNKS = IDX_PER_W // IDX_CHUNK
LANES = 16
E_VECS = E_SIZE // LANES

VB = 4096
NB = 24
V_MAIN = VB * NB
V_TAIL = V_SIZE - V_MAIN
NC = 4
RB = BATCH // NC


def _sc_gather_mean(idx_flat, emb_table):
    mesh = plsc.VectorSubcoreMesh(core_axis_name="c", subcore_axis_name="s")

    @functools.partial(
        pl.kernel,
        out_type=jax.ShapeDtypeStruct((BATCH, E_SIZE), jnp.float32),
        mesh=mesh,
        compiler_params=pltpu.CompilerParams(use_tc_tiling_on_sc=False),
        scratch_types=[
            pltpu.VMEM((N_CHUNKS, IDX_CHUNK), jnp.int32),
            pltpu.VMEM((IDX_PER_W, E_SIZE), jnp.float32),
            pltpu.VMEM((B_PER_W, E_SIZE), jnp.float32),
            pltpu.SemaphoreType.DMA,
        ],
    )
    def gather_mean(idx_hbm, table_hbm, out_hbm, idx_v, rows_v, acc_v, sem):
        wid = lax.axis_index("s") * 2 + lax.axis_index("c")
        pltpu.sync_copy(idx_hbm.at[wid], idx_v)
        copies = []
        for j in range(N_CHUNKS):
            copies.append(
                pltpu.async_copy(
                    table_hbm.at[idx_v.at[j]],
                    rows_v.at[pl.ds(j * IDX_CHUNK, IDX_CHUNK)],
                    sem,
                )
            )
        for c in copies:
            c.wait()

        def pool_one(s, carry):
            for e in range(E_VECS):
                acc = rows_v[s * HIST, pl.ds(e * LANES, LANES)]
                for h in range(1, HIST):
                    acc = acc + rows_v[s * HIST + h, pl.ds(e * LANES, LANES)]
                acc_v[s, pl.ds(e * LANES, LANES)] = acc * (1.0 / HIST)
            return carry

        lax.fori_loop(0, B_PER_W, pool_one, 0)
        pltpu.sync_copy(acc_v, out_hbm.at[pl.ds(wid * B_PER_W, B_PER_W)])

    return gather_mean(idx_flat, emb_table)


def _tc_matmul(h, lin_w):
    vb = 4096

    def mm(h_ref, w_ref, o_ref):
        o_ref[...] = lax.dot_general(
            h_ref[...], w_ref[...],
            (((1,), (1,)), ((), ())),
            preferred_element_type=jnp.float32,
        )

    return pl.pallas_call(
        mm,
        grid=(pl.cdiv(V_SIZE, vb),),
        in_specs=[
            pl.BlockSpec((BATCH, E_SIZE), lambda i: (0, 0)),
            pl.BlockSpec((vb, E_SIZE), lambda i: (i, 0)),
        ],
        out_specs=pl.BlockSpec((BATCH, vb), lambda i: (0, i)),
        out_shape=jax.ShapeDtypeStruct((BATCH, V_SIZE), jnp.float32),
        compiler_params=pltpu.CompilerParams(
            dimension_semantics=("arbitrary",),
            vmem_limit_bytes=100 * 1024 * 1024,
        ),
    )(h, lin_w)


def kernel(input, emb_table, lin_w):
    idx_flat = input.reshape(NUM_WORKERS, N_CHUNKS, IDX_CHUNK)
    h = _sc_gather_mean(idx_flat, emb_table)
    return _tc_matmul(h, lin_w)


# speedup vs baseline: 1.0094x; 1.0094x over previous
import functools

import jax
import jax.numpy as jnp
from jax import lax
from jax.experimental import pallas as pl
from jax.experimental.pallas import tpu as pltpu
from jax.experimental.pallas import tpu_sc as plsc

V_SIZE = 100000
E_SIZE = 64
BATCH = 1024
HIST = 20

NUM_WORKERS = 32
B_PER_W = BATCH // NUM_WORKERS
IDX_PER_W = B_PER_W * HIST
IDX_CHUNK = 128
N_CHUNKS = IDX_PER_W // IDX_CHUNK
LANES = 16
E_VECS = E_SIZE // LANES

VB = 4096
NB = 24
V_MAIN = VB * NB
V_TAIL = V_SIZE - V_MAIN
NC = 4
RB = BATCH // NC


def _sc_gather_mean(idx_flat, emb_table):
    mesh = plsc.VectorSubcoreMesh(core_axis_name="c", subcore_axis_name="s")

    @functools.partial(
        pl.kernel,
        out_type=jax.ShapeDtypeStruct((BATCH, E_SIZE), jnp.float32),
        mesh=mesh,
        compiler_params=pltpu.CompilerParams(use_tc_tiling_on_sc=False),
        scratch_types=[
            pltpu.VMEM((N_CHUNKS, IDX_CHUNK), jnp.int32),
            pltpu.VMEM((IDX_PER_W, E_SIZE), jnp.float32),
            pltpu.VMEM((B_PER_W, E_SIZE), jnp.float32),
            pltpu.SemaphoreType.DMA,
        ],
    )
    def gather_mean(idx_hbm, table_hbm, out_hbm, idx_v, rows_v, acc_v, sem):
        wid = lax.axis_index("s") * 2 + lax.axis_index("c")
        pltpu.sync_copy(idx_hbm.at[wid], idx_v)
        copies = []
        for j in range(N_CHUNKS):
            copies.append(
                pltpu.async_copy(
                    table_hbm.at[idx_v.at[j]],
                    rows_v.at[pl.ds(j * IDX_CHUNK, IDX_CHUNK)],
                    sem,
                )
            )
        for c in copies:
            c.wait()

        def pool_one(s, carry):
            for e in range(E_VECS):
                acc = rows_v[s * HIST, pl.ds(e * LANES, LANES)]
                for h in range(1, HIST):
                    acc = acc + rows_v[s * HIST + h, pl.ds(e * LANES, LANES)]
                acc_v[s, pl.ds(e * LANES, LANES)] = acc * (1.0 / HIST)
            return carry

        lax.fori_loop(0, B_PER_W, pool_one, 0)
        pltpu.sync_copy(acc_v, out_hbm.at[pl.ds(wid * B_PER_W, B_PER_W)])

    return gather_mean(idx_flat, emb_table)


def _tc_matmul(h, lin_w):
    vb = 4096

    def mm(h_ref, w_ref, o_ref):
        o_ref[...] = lax.dot_general(
            h_ref[...], w_ref[...],
            (((1,), (1,)), ((), ())),
            preferred_element_type=jnp.float32,
        )

    return pl.pallas_call(
        mm,
        grid=(pl.cdiv(V_SIZE, vb),),
        in_specs=[
            pl.BlockSpec((BATCH, E_SIZE), lambda i: (0, 0)),
            pl.BlockSpec((vb, E_SIZE), lambda i: (i, 0)),
        ],
        out_specs=pl.BlockSpec((BATCH, vb), lambda i: (0, i)),
        out_shape=jax.ShapeDtypeStruct((BATCH, V_SIZE), jnp.float32),
        compiler_params=pltpu.CompilerParams(
            dimension_semantics=("arbitrary",),
            vmem_limit_bytes=100 * 1024 * 1024,
        ),
    )(h, lin_w)


def kernel(input, emb_table, lin_w):
    idx_flat = input.reshape(NUM_WORKERS, N_CHUNKS, IDX_CHUNK)
    h = _sc_gather_mean(idx_flat, emb_table)
    return _tc_matmul(h.astype(jnp.bfloat16), lin_w.astype(jnp.bfloat16))
